# Initial kernel scaffold; baseline (speedup 1.0000x reference)
import jax
import jax.numpy as jnp
from jax.experimental import pallas as pl


def kernel(hidden_states, gate_w, gate_up_w, down_w):
    raise NotImplementedError("write your pallas kernel here")


# speedup vs baseline: 2.5650x; 2.5650x over previous
import functools

import jax
import jax.numpy as jnp
from jax.experimental import pallas as pl
from jax.experimental.pallas import tpu as pltpu

M, H, I, E = 2048, 1024, 768, 8
TOP_K = 2


def _router_weights(x, gate_w):
    logits = jax.lax.dot_general(
        x, gate_w, (((1,), (1,)), ((), ())),
        preferred_element_type=jnp.float32)
    p = jax.nn.softmax(logits, axis=-1)
    col = jax.lax.broadcasted_iota(jnp.int32, p.shape, 1)
    big = jnp.int32(E + 1)
    m1 = jnp.max(p, axis=1, keepdims=True)
    a1 = jnp.min(jnp.where(p == m1, col, big), axis=1, keepdims=True)
    sel1 = col == a1
    pm = jnp.where(sel1, -jnp.inf, p)
    m2 = jnp.max(pm, axis=1, keepdims=True)
    a2 = jnp.min(jnp.where(pm == m2, col, big), axis=1, keepdims=True)
    sel2 = col == a2
    denom = m1 + m2
    return (jnp.where(sel1, m1, 0.0) + jnp.where(sel2, m2, 0.0)) / denom


def _moe_body(x_ref, gate_w_ref, gup_ref, down_ref, out_ref, w_scr):
    e = pl.program_id(0)

    @pl.when(e == 0)
    def _():
        w_scr[...] = _router_weights(x_ref[...], gate_w_ref[...])

    x = x_ref[...]
    gu = jax.lax.dot_general(
        x, gup_ref[0], (((1,), (0,)), ((), ())),
        preferred_element_type=jnp.float32)
    g = gu[:, :I]
    u = gu[:, I:]
    h = (g / (1.0 + jnp.exp(-g))) * u
    y = jax.lax.dot_general(
        h, down_ref[0], (((1,), (0,)), ((), ())),
        preferred_element_type=jnp.float32)
    w_all = w_scr[...]
    col = jax.lax.broadcasted_iota(jnp.int32, w_all.shape, 1)
    w_e = jnp.sum(jnp.where(col == e, w_all, 0.0), axis=1, keepdims=True)
    contrib = y * w_e

    @pl.when(e == 0)
    def _():
        out_ref[...] = contrib

    @pl.when(e != 0)
    def _():
        out_ref[...] = out_ref[...] + contrib


@jax.jit
def kernel(hidden_states, gate_w, gate_up_w, down_w):
    return pl.pallas_call(
        _moe_body,
        grid=(E,),
        in_specs=[
            pl.BlockSpec((M, H), lambda e: (0, 0)),
            pl.BlockSpec((E, H), lambda e: (0, 0)),
            pl.BlockSpec((1, H, 2 * I), lambda e: (e, 0, 0)),
            pl.BlockSpec((1, I, H), lambda e: (e, 0, 0)),
        ],
        out_specs=pl.BlockSpec((M, H), lambda e: (0, 0)),
        out_shape=jax.ShapeDtypeStruct((M, H), jnp.float32),
        scratch_shapes=[pltpu.VMEM((M, E), jnp.float32)],
    )(hidden_states, gate_w, gate_up_w, down_w)
